# Initial kernel scaffold; baseline (speedup 1.0000x reference)
import jax
import jax.numpy as jnp
from jax.experimental import pallas as pl


def kernel(features):
    raise NotImplementedError("write your pallas kernel here")


# speedup vs baseline: 1.3558x; 1.3558x over previous
import jax
import jax.numpy as jnp
import numpy as np
from jax import lax
from jax.experimental import pallas as pl
from jax.experimental.pallas import tpu as pltpu
from jax.experimental.pallas import tpu_sc as plsc

_CONF_DELTA = 0.05
_NUM_OBJ_CLASSES = 42

_N_ROWS = 262144
_N_FEATS = 204

_NC = 2
_NS = 16
_NW = _NC * _NS

_ROWS_PER_W = _N_ROWS // _NW
_CHUNK = 128
_NBUF = 4
_NCHUNKS = _ROWS_PER_W // _CHUNK
_NGROUPS = _NCHUNKS // _NBUF
_LANES = 16


def _activation_cols(D):
    num_obj_feats = _NUM_OBJ_CLASSES - 2
    num_obj_points = num_obj_feats * 2
    obj_acts_idx = num_obj_points + 1 + num_obj_points + 2 + 1
    return [0, num_obj_points + 1] + list(range(obj_acts_idx, D))


_COLS = _activation_cols(_N_FEATS)
_NIDX = len(_COLS)
_PAD = 3 * _LANES - _NIDX
_COLS_PADDED = np.asarray(_COLS + [_COLS[-1]] * _PAD, np.int32).reshape(3, _LANES)
_VALID = [min(_LANES, max(0, _NIDX - j * _LANES)) for j in range(3)]


_CELEM = _CHUNK * _N_FEATS


def _sc_body(feat_hbm, dvec_hbm, cols_hbm, out_hbm,
             bufs, dvec_v, cols_v, sems_in, sems_out):
    wid = lax.axis_index("c") * _NS + lax.axis_index("s")
    base = wid * _ROWS_PER_W * _N_FEATS

    pltpu.sync_copy(dvec_hbm, dvec_v)
    pltpu.sync_copy(cols_hbm, cols_v)
    dv = dvec_v[...]
    col_vecs = [cols_v[j] for j in range(3)]
    masks = [lax.iota(jnp.int32, _LANES) < _VALID[j] for j in range(3)]

    def start_in(b, chunk):
        pltpu.async_copy(
            feat_hbm.at[pl.ds(base + chunk * _CELEM, _CELEM)],
            bufs[b], sems_in[b])

    def wait_in(b):
        pltpu.make_async_copy(
            feat_hbm.at[pl.ds(base, _CELEM)], bufs[b], sems_in[b]).wait()

    def start_out(b, chunk):
        pltpu.async_copy(
            bufs[b], out_hbm.at[pl.ds(base + chunk * _CELEM, _CELEM)],
            sems_out[b])

    def wait_out(b):
        pltpu.make_async_copy(
            bufs[b], out_hbm.at[pl.ds(base, _CELEM)], sems_out[b]).wait()

    def compute(b):
        buf = bufs[b]

        def row_body(r, carry):
            r0 = r * _N_FEATS
            for j in range(3):
                m = masks[j]
                idx = col_vecs[j] + r0
                v = plsc.load_gather(buf, [idx], mask=m)
                t = jnp.minimum(jnp.maximum(v + dv, 0.0), 1.0)
                u = jnp.where(v != 0.0, t, v)
                plsc.store_scatter(buf, [idx], u, mask=m)
            return carry

        lax.fori_loop(0, _CHUNK, row_body, 0)

    for b in range(_NBUF):
        start_in(b, b)

    def group_body(g, carry):
        for b in range(_NBUF):
            i = g * _NBUF + b
            wait_in(b)
            compute(b)
            start_out(b, i)
            br = (b - 2) % _NBUF
            c = i + _NBUF - 2

            @pl.when(jnp.logical_and(c >= _NBUF, c < _NCHUNKS))
            def _():
                wait_out(br)
                start_in(br, c)

        return carry

    lax.fori_loop(0, _NGROUPS, group_body, 0)

    for b in range(_NBUF):
        wait_out(b)


@jax.jit
def kernel(features):
    delta = jax.random.uniform(
        jax.random.key(1), (), dtype=jnp.float32,
        minval=-_CONF_DELTA, maxval=_CONF_DELTA,
    )
    dvec = jnp.full((_LANES,), delta, jnp.float32)
    cols = jnp.asarray(_COLS_PADDED)

    mesh = plsc.VectorSubcoreMesh(
        core_axis_name="c", subcore_axis_name="s",
        num_cores=_NC, num_subcores=_NS)

    run = pl.kernel(
        _sc_body,
        out_type=jax.ShapeDtypeStruct((_N_ROWS * _N_FEATS,), jnp.float32),
        mesh=mesh,
        compiler_params=pltpu.CompilerParams(
            use_tc_tiling_on_sc=False, needs_layout_passes=False),
        scratch_types=dict(
            bufs=[pltpu.VMEM((_CELEM,), jnp.float32)
                  for _ in range(_NBUF)],
            dvec_v=pltpu.VMEM((_LANES,), jnp.float32),
            cols_v=pltpu.VMEM((3, _LANES), jnp.int32),
            sems_in=[pltpu.SemaphoreType.DMA for _ in range(_NBUF)],
            sems_out=[pltpu.SemaphoreType.DMA for _ in range(_NBUF)],
        ),
    )
    out = run(features.reshape(-1), dvec, cols)
    return out.reshape(_N_ROWS, _N_FEATS)


# speedup vs baseline: 2.7452x; 2.0248x over previous
import jax
import jax.numpy as jnp
import numpy as np
from jax import lax
from jax.experimental import pallas as pl
from jax.experimental.pallas import tpu as pltpu
from jax.experimental.pallas import tpu_sc as plsc

_CONF_DELTA = 0.05
_NUM_OBJ_CLASSES = 42

_N_ROWS = 262144
_N_FEATS = 204

_NC = 2
_NS = 16
_NW = _NC * _NS

_ROWS_PER_W = _N_ROWS // _NW
_CHUNK = 64
_NBUF = 4
_NCHUNKS = _ROWS_PER_W // _CHUNK
_NGROUPS = _NCHUNKS // _NBUF
_LANES = 16


def _activation_cols(D):
    num_obj_feats = _NUM_OBJ_CLASSES - 2
    num_obj_points = num_obj_feats * 2
    obj_acts_idx = num_obj_points + 1 + num_obj_points + 2 + 1
    return [0, num_obj_points + 1] + list(range(obj_acts_idx, D))


_COLS = _activation_cols(_N_FEATS)
_NIDX = len(_COLS)
_PAD = 3 * _LANES - _NIDX
_COLS_PADDED = np.asarray(_COLS + [_COLS[-1]] * _PAD, np.int32).reshape(3, _LANES)
_VALID = [min(_LANES, max(0, _NIDX - j * _LANES)) for j in range(3)]


def _sc_body(feat_hbm, dvec_hbm, cols_hbm, out_hbm,
             bufs, dvec_v, cols_v, sems_in, sems_out):
    wid = lax.axis_index("c") * _NS + lax.axis_index("s")
    row0 = wid * _ROWS_PER_W

    pltpu.sync_copy(dvec_hbm, dvec_v)
    pltpu.sync_copy(cols_hbm, cols_v)
    dv = dvec_v[...]
    col_vecs = [cols_v[j] for j in range(3)]
    masks = [lax.iota(jnp.int32, _LANES) < _VALID[j] for j in range(3)]

    def start_in(b, chunk):
        pltpu.async_copy(
            feat_hbm.at[pl.ds(row0 + chunk * _CHUNK, _CHUNK), :],
            bufs[b], sems_in[b])

    def wait_in(b):
        pltpu.make_async_copy(
            feat_hbm.at[pl.ds(row0, _CHUNK), :], bufs[b], sems_in[b]).wait()

    def start_out(b, chunk):
        pltpu.async_copy(
            bufs[b], out_hbm.at[pl.ds(row0 + chunk * _CHUNK, _CHUNK), :],
            sems_out[b])

    def wait_out(b):
        pltpu.make_async_copy(
            bufs[b], out_hbm.at[pl.ds(row0, _CHUNK), :], sems_out[b]).wait()

    def compute(b):
        buf = bufs[b]

        @plsc.parallel_loop(0, _CHUNK, 1, unroll=2)
        def row_body(r):
            rvec = jnp.full((_LANES,), r, jnp.int32)
            vs = [plsc.load_gather(buf, [rvec, col_vecs[j]], mask=masks[j])
                  for j in range(3)]
            for j in range(3):
                v = vs[j]
                t = jnp.minimum(jnp.maximum(v + dv, 0.0), 1.0)
                u = jnp.where(v == 0.0, v, t)
                plsc.store_scatter(buf, [rvec, col_vecs[j]], u, mask=masks[j])

    for b in range(_NBUF):
        start_in(b, b)

    def group_body(g, carry):
        for b in range(_NBUF):
            i = g * _NBUF + b
            wait_in(b)
            compute(b)
            start_out(b, i)
            br = (b - 2) % _NBUF
            c = i + _NBUF - 2

            @pl.when(jnp.logical_and(c >= _NBUF, c < _NCHUNKS))
            def _():
                wait_out(br)
                start_in(br, c)

        return carry

    lax.fori_loop(0, _NGROUPS, group_body, 0)

    for b in range(_NBUF):
        wait_out(b)


@jax.jit
def kernel(features):
    delta = jax.random.uniform(
        jax.random.key(1), (), dtype=jnp.float32,
        minval=-_CONF_DELTA, maxval=_CONF_DELTA,
    )
    dvec = jnp.full((_LANES,), delta, jnp.float32)
    cols = jnp.asarray(_COLS_PADDED)

    mesh = plsc.VectorSubcoreMesh(
        core_axis_name="c", subcore_axis_name="s",
        num_cores=_NC, num_subcores=_NS)

    run = pl.kernel(
        _sc_body,
        out_type=jax.ShapeDtypeStruct((_N_ROWS, _N_FEATS), jnp.float32),
        mesh=mesh,
        compiler_params=pltpu.CompilerParams(needs_layout_passes=False),
        scratch_types=dict(
            bufs=[pltpu.VMEM((_CHUNK, _N_FEATS), jnp.float32)
                  for _ in range(_NBUF)],
            dvec_v=pltpu.VMEM((_LANES,), jnp.float32),
            cols_v=pltpu.VMEM((3, _LANES), jnp.int32),
            sems_in=[pltpu.SemaphoreType.DMA for _ in range(_NBUF)],
            sems_out=[pltpu.SemaphoreType.DMA for _ in range(_NBUF)],
        ),
    )
    return run(features, dvec, cols)


# speedup vs baseline: 2.7581x; 1.0047x over previous
import jax
import jax.numpy as jnp
import numpy as np
from jax import lax
from jax.experimental import pallas as pl
from jax.experimental.pallas import tpu as pltpu
from jax.experimental.pallas import tpu_sc as plsc

_CONF_DELTA = 0.05
_NUM_OBJ_CLASSES = 42

_N_ROWS = 262144
_N_FEATS = 204

_NC = 2
_NS = 16
_NW = _NC * _NS

_ROWS_PER_W = _N_ROWS // _NW
_CHUNK = 32
_NBUF = 8
_NCHUNKS = _ROWS_PER_W // _CHUNK
_NGROUPS = _NCHUNKS // _NBUF
_LANES = 16


def _activation_cols(D):
    num_obj_feats = _NUM_OBJ_CLASSES - 2
    num_obj_points = num_obj_feats * 2
    obj_acts_idx = num_obj_points + 1 + num_obj_points + 2 + 1
    return [0, num_obj_points + 1] + list(range(obj_acts_idx, D))


_COLS = _activation_cols(_N_FEATS)
_NIDX = len(_COLS)
_PAD = 3 * _LANES - _NIDX
_COLS_PADDED = np.asarray(_COLS + [_COLS[-1]] * _PAD, np.int32).reshape(3, _LANES)
_VALID = [min(_LANES, max(0, _NIDX - j * _LANES)) for j in range(3)]


def _sc_body(feat_hbm, dvec_hbm, cols_hbm, out_hbm,
             bufs, dvec_v, cols_v, sems_in, sems_out):
    wid = lax.axis_index("c") * _NS + lax.axis_index("s")
    row0 = wid * _ROWS_PER_W

    pltpu.sync_copy(dvec_hbm, dvec_v)
    pltpu.sync_copy(cols_hbm, cols_v)
    dv = dvec_v[...]
    col_vecs = [cols_v[j] for j in range(3)]
    masks = [lax.iota(jnp.int32, _LANES) < _VALID[j] for j in range(3)]

    def start_in(b, chunk):
        pltpu.async_copy(
            feat_hbm.at[pl.ds(row0 + chunk * _CHUNK, _CHUNK), :],
            bufs[b], sems_in[b])

    def wait_in(b):
        pltpu.make_async_copy(
            feat_hbm.at[pl.ds(row0, _CHUNK), :], bufs[b], sems_in[b]).wait()

    def start_out(b, chunk):
        pltpu.async_copy(
            bufs[b], out_hbm.at[pl.ds(row0 + chunk * _CHUNK, _CHUNK), :],
            sems_out[b])

    def wait_out(b):
        pltpu.make_async_copy(
            bufs[b], out_hbm.at[pl.ds(row0, _CHUNK), :], sems_out[b]).wait()

    def compute(b):
        buf = bufs[b]

        @plsc.parallel_loop(0, _CHUNK, 1, unroll=2)
        def row_body(r):
            rvec = jnp.full((_LANES,), r, jnp.int32)
            vs = [plsc.load_gather(buf, [rvec, col_vecs[j]], mask=masks[j])
                  for j in range(3)]
            for j in range(3):
                v = vs[j]
                t = jnp.minimum(jnp.maximum(v + dv, 0.0), 1.0)
                u = jnp.where(v == 0.0, v, t)
                plsc.store_scatter(buf, [rvec, col_vecs[j]], u, mask=masks[j])

    for b in range(_NBUF):
        start_in(b, b)

    def group_body(g, carry):
        for b in range(_NBUF):
            i = g * _NBUF + b
            wait_in(b)
            compute(b)
            start_out(b, i)
            br = (b - 2) % _NBUF
            c = i + _NBUF - 2

            @pl.when(jnp.logical_and(c >= _NBUF, c < _NCHUNKS))
            def _():
                wait_out(br)
                start_in(br, c)

        return carry

    lax.fori_loop(0, _NGROUPS, group_body, 0)

    for b in range(_NBUF):
        wait_out(b)


@jax.jit
def kernel(features):
    delta = jax.random.uniform(
        jax.random.key(1), (), dtype=jnp.float32,
        minval=-_CONF_DELTA, maxval=_CONF_DELTA,
    )
    dvec = jnp.full((_LANES,), delta, jnp.float32)
    cols = jnp.asarray(_COLS_PADDED)

    mesh = plsc.VectorSubcoreMesh(
        core_axis_name="c", subcore_axis_name="s",
        num_cores=_NC, num_subcores=_NS)

    run = pl.kernel(
        _sc_body,
        out_type=jax.ShapeDtypeStruct((_N_ROWS, _N_FEATS), jnp.float32),
        mesh=mesh,
        compiler_params=pltpu.CompilerParams(needs_layout_passes=False),
        scratch_types=dict(
            bufs=[pltpu.VMEM((_CHUNK, _N_FEATS), jnp.float32)
                  for _ in range(_NBUF)],
            dvec_v=pltpu.VMEM((_LANES,), jnp.float32),
            cols_v=pltpu.VMEM((3, _LANES), jnp.int32),
            sems_in=[pltpu.SemaphoreType.DMA for _ in range(_NBUF)],
            sems_out=[pltpu.SemaphoreType.DMA for _ in range(_NBUF)],
        ),
    )
    return run(features, dvec, cols)
